# baseline (speedup 1.0000x reference)
import functools

import jax
import jax.numpy as jnp
from jax import lax
from jax.experimental import pallas as pl
from jax.experimental.pallas import tpu as pltpu
from jax.experimental.pallas import tpu_sc as plsc


def kernel(inputs, pos_embedding):
    B, S, D = inputs.shape

    info = plsc.get_sparse_core_info()
    NC, NS, L = info.num_cores, info.num_subcores, info.num_lanes
    NW = NC * NS
    rows_w = S // NW
    CH = 8
    NCH = rows_w // CH
    GPR = D // L
    NB = 3

    mesh = plsc.VectorSubcoreMesh(core_axis_name="c", subcore_axis_name="s")

    @functools.partial(
        pl.kernel,
        mesh=mesh,
        out_type=jax.ShapeDtypeStruct((B, S, D), jnp.float32),
        scratch_types=(
            [pltpu.VMEM((CH, D), jnp.float32)] * NB
            + [pltpu.VMEM((B, CH, D), jnp.float32)] * NB
            + [pltpu.SemaphoreType.DMA] * (2 * NB)
        ),
    )
    def k(x_hbm, p_hbm, o_hbm, *scr):
        p_bufs = scr[:NB]
        x_bufs = scr[NB : 2 * NB]
        in_sems = scr[2 * NB : 3 * NB]
        out_sems = scr[3 * NB :]
        wid = lax.axis_index("s") * NC + lax.axis_index("c")
        base = wid * rows_w

        def in_copies(ci):
            s = ci % NB
            r0 = base + ci * CH
            return [
                pltpu.make_async_copy(p_hbm.at[pl.ds(r0, CH)], p_bufs[s], in_sems[s]),
                pltpu.make_async_copy(x_hbm.at[:, pl.ds(r0, CH)], x_bufs[s], in_sems[s]),
            ]

        def out_copies(ci):
            s = ci % NB
            r0 = base + ci * CH
            return [
                pltpu.make_async_copy(x_bufs[s], o_hbm.at[:, pl.ds(r0, CH)], out_sems[s])
            ]

        for ci in range(min(2, NCH)):
            for c in in_copies(ci):
                c.start()

        for ci in range(NCH):
            s = ci % NB
            for c in in_copies(ci):
                c.wait()

            pb = p_bufs[s]
            xb = x_bufs[s]

            @plsc.parallel_loop(0, CH * GPR, unroll=8)
            def _grp(g):
                r = lax.shift_right_logical(g, 6)
                go = (g & (GPR - 1)) * L
                pv = pb[r, pl.ds(go, L)]
                for b in range(B):
                    xb[b, r, pl.ds(go, L)] = xb[b, r, pl.ds(go, L)] + pv

            for c in out_copies(ci):
                c.start()
            if ci + 2 < NCH:
                if ci - 2 >= 0:
                    for c in out_copies(ci - 2):
                        c.wait()
                for c in in_copies(ci + 2):
                    c.start()

        for ci in range(max(0, NCH - 4), NCH):
            for c in out_copies(ci):
                c.wait()

    return k(inputs, pos_embedding)


# speedup vs baseline: 1.0669x; 1.0669x over previous
import functools

import jax
import jax.numpy as jnp
from jax import lax
from jax.experimental import pallas as pl
from jax.experimental.pallas import tpu as pltpu
from jax.experimental.pallas import tpu_sc as plsc


def _add_body(x_ref, p_ref, o_ref):
    o_ref[...] = x_ref[...] + p_ref[...]


def _merge_body(src_ref, canvas_ref, o_ref):
    o_ref[...] = src_ref[...]


def _sc_part(inputs, pos_embedding, row0, nrows):
    B, S, D = inputs.shape

    info = plsc.get_sparse_core_info()
    NC, NS, L = info.num_cores, info.num_subcores, info.num_lanes
    NW = NC * NS
    rows_w = nrows // NW
    CH = 8
    NCH = rows_w // CH
    GPR = D // L
    NB = 3

    mesh = plsc.VectorSubcoreMesh(core_axis_name="c", subcore_axis_name="s")

    @functools.partial(
        pl.kernel,
        mesh=mesh,
        out_type=jax.ShapeDtypeStruct((B, nrows, D), jnp.float32),
        scratch_types=(
            [pltpu.VMEM((CH, D), jnp.float32)] * NB
            + [pltpu.VMEM((B, CH, D), jnp.float32)] * NB
            + [pltpu.SemaphoreType.DMA] * (2 * NB)
        ),
    )
    def k(x_hbm, p_hbm, o_hbm, *scr):
        p_bufs = scr[:NB]
        x_bufs = scr[NB : 2 * NB]
        in_sems = scr[2 * NB : 3 * NB]
        out_sems = scr[3 * NB :]
        wid = lax.axis_index("s") * NC + lax.axis_index("c")
        lbase = wid * rows_w

        def in_copies(ci):
            s = ci % NB
            r0 = row0 + lbase + ci * CH
            return [
                pltpu.make_async_copy(p_hbm.at[pl.ds(r0, CH)], p_bufs[s], in_sems[s]),
                pltpu.make_async_copy(x_hbm.at[:, pl.ds(r0, CH)], x_bufs[s], in_sems[s]),
            ]

        def out_copies(ci):
            s = ci % NB
            r0 = lbase + ci * CH
            return [
                pltpu.make_async_copy(x_bufs[s], o_hbm.at[:, pl.ds(r0, CH)], out_sems[s])
            ]

        for ci in range(min(2, NCH)):
            for c in in_copies(ci):
                c.start()

        for ci in range(NCH):
            s = ci % NB
            for c in in_copies(ci):
                c.wait()

            pb = p_bufs[s]
            xb = x_bufs[s]

            @plsc.parallel_loop(0, CH * GPR, unroll=8)
            def _grp(g):
                r = lax.shift_right_logical(g, 6)
                go = (g & (GPR - 1)) * L
                pv = pb[r, pl.ds(go, L)]
                for b in range(B):
                    xb[b, r, pl.ds(go, L)] = xb[b, r, pl.ds(go, L)] + pv

            for c in out_copies(ci):
                c.start()
            if ci + 2 < NCH:
                if ci - 1 >= 0:
                    for c in out_copies(ci - 1):
                        c.wait()
                for c in in_copies(ci + 2):
                    c.start()

        for ci in range(max(0, NCH - 3), NCH):
            for c in out_copies(ci):
                c.wait()

    return k(inputs, pos_embedding)


def kernel(inputs, pos_embedding):
    B, S, D = inputs.shape
    S_TC = 3584
    S_SC = S - S_TC

    sc_out = _sc_part(inputs, pos_embedding, S_TC, S_SC)

    BS = 512
    canvas = pl.pallas_call(
        _add_body,
        grid=(S_TC // BS,),
        in_specs=[
            pl.BlockSpec((B, BS, D), lambda i: (0, i, 0)),
            pl.BlockSpec((BS, D), lambda i: (i, 0)),
        ],
        out_specs=pl.BlockSpec((B, BS, D), lambda i: (0, i, 0)),
        out_shape=jax.ShapeDtypeStruct((B, S, D), inputs.dtype),
    )(inputs, pos_embedding)

    off = S_TC // BS
    return pl.pallas_call(
        _merge_body,
        grid=(S_SC // BS,),
        in_specs=[
            pl.BlockSpec((B, BS, D), lambda i: (0, i, 0)),
            pl.BlockSpec(memory_space=pl.ANY),
        ],
        out_specs=pl.BlockSpec((B, BS, D), lambda i: (0, off + i, 0)),
        out_shape=jax.ShapeDtypeStruct((B, S, D), inputs.dtype),
        input_output_aliases={1: 0},
    )(sc_out, canvas)


# speedup vs baseline: 1.1222x; 1.0518x over previous
import functools

import jax
import jax.numpy as jnp
from jax import lax
from jax.experimental import pallas as pl
from jax.experimental.pallas import tpu as pltpu
from jax.experimental.pallas import tpu_sc as plsc


def _add_body(x_ref, p_ref, o_ref):
    o_ref[...] = x_ref[...] + p_ref[...]


def _merge_body(src_ref, canvas_ref, o_ref):
    o_ref[...] = src_ref[...]


def _sc_part(inputs, pos_embedding, row0, nrows):
    B, S, D = inputs.shape

    info = plsc.get_sparse_core_info()
    NC, NS, L = info.num_cores, info.num_subcores, info.num_lanes
    NW = NC * NS
    rows_w = nrows // NW
    CH = 8
    NCH = rows_w // CH
    GPR = D // L
    NB = 3

    mesh = plsc.VectorSubcoreMesh(core_axis_name="c", subcore_axis_name="s")

    @functools.partial(
        pl.kernel,
        mesh=mesh,
        out_type=jax.ShapeDtypeStruct((B, nrows, D), jnp.float32),
        scratch_types=(
            [pltpu.VMEM((CH, D), jnp.float32)] * NB
            + [pltpu.VMEM((B, CH, D), jnp.float32)] * NB
            + [pltpu.SemaphoreType.DMA] * (2 * NB)
        ),
    )
    def k(x_hbm, p_hbm, o_hbm, *scr):
        p_bufs = scr[:NB]
        x_bufs = scr[NB : 2 * NB]
        in_sems = scr[2 * NB : 3 * NB]
        out_sems = scr[3 * NB :]
        wid = lax.axis_index("s") * NC + lax.axis_index("c")
        lbase = wid * rows_w

        def in_copies(ci):
            s = ci % NB
            r0 = row0 + lbase + ci * CH
            return [
                pltpu.make_async_copy(p_hbm.at[pl.ds(r0, CH)], p_bufs[s], in_sems[s]),
                pltpu.make_async_copy(x_hbm.at[:, pl.ds(r0, CH)], x_bufs[s], in_sems[s]),
            ]

        def out_copies(ci):
            s = ci % NB
            r0 = lbase + ci * CH
            return [
                pltpu.make_async_copy(x_bufs[s], o_hbm.at[:, pl.ds(r0, CH)], out_sems[s])
            ]

        for ci in range(min(2, NCH)):
            for c in in_copies(ci):
                c.start()

        for ci in range(NCH):
            s = ci % NB
            for c in in_copies(ci):
                c.wait()

            pb = p_bufs[s]
            xb = x_bufs[s]

            @plsc.parallel_loop(0, CH * GPR, unroll=8)
            def _grp(g):
                r = lax.shift_right_logical(g, 6)
                go = (g & (GPR - 1)) * L
                pv = pb[r, pl.ds(go, L)]
                for b in range(B):
                    xb[b, r, pl.ds(go, L)] = xb[b, r, pl.ds(go, L)] + pv

            for c in out_copies(ci):
                c.start()
            if ci + 2 < NCH:
                if ci - 1 >= 0:
                    for c in out_copies(ci - 1):
                        c.wait()
                for c in in_copies(ci + 2):
                    c.start()

        for ci in range(max(0, NCH - 3), NCH):
            for c in out_copies(ci):
                c.wait()

    return k(inputs, pos_embedding)


def kernel(inputs, pos_embedding):
    B, S, D = inputs.shape
    S_TC = 3840
    S_SC = S - S_TC

    sc_out = _sc_part(inputs, pos_embedding, S_TC, S_SC)

    BS = 640
    canvas = pl.pallas_call(
        _add_body,
        grid=(S_TC // BS,),
        in_specs=[
            pl.BlockSpec((B, BS, D), lambda i: (0, i, 0)),
            pl.BlockSpec((BS, D), lambda i: (i, 0)),
        ],
        out_specs=pl.BlockSpec((B, BS, D), lambda i: (0, i, 0)),
        out_shape=jax.ShapeDtypeStruct((B, S, D), inputs.dtype),
    )(inputs, pos_embedding)

    BM = S_SC
    off = S_TC // BM
    return pl.pallas_call(
        _merge_body,
        grid=(S_SC // BM,),
        in_specs=[
            pl.BlockSpec((B, BM, D), lambda i: (0, i, 0)),
            pl.BlockSpec(memory_space=pl.ANY),
        ],
        out_specs=pl.BlockSpec((B, BM, D), lambda i: (0, off + i, 0)),
        out_shape=jax.ShapeDtypeStruct((B, S, D), inputs.dtype),
        input_output_aliases={1: 0},
    )(sc_out, canvas)
